# baseline (speedup 1.0000x reference)
import functools

import jax
import jax.numpy as jnp
from jax import lax
from jax.experimental import pallas as pl
from jax.experimental.pallas import tpu as pltpu
from jax.experimental.pallas import tpu_sc as plsc

N_ATOMS = 100000
N_IN = 128
N_HIDDEN = 64
N_MOL = 1024

LANES = 16
NWORKERS = 32
BLK = 20480
GRID = 5
N_PAD = GRID * BLK
CHUNK = N_PAD // NWORKERS
TAIL = N_ATOMS - (NWORKERS - 1) * CHUNK
CROWS = CHUNK // 128
ROWS = BLK // 128


def _mlp_body(x_ref, w1t_ref, b1_ref, w2_ref, b2_ref, y_ref):
    i = pl.program_id(0)
    h = lax.dot_general(
        x_ref[...].astype(jnp.bfloat16),
        w1t_ref[...].astype(jnp.bfloat16),
        (((1,), (1,)), ((), ())),
        preferred_element_type=jnp.float32,
    )
    h = h + b1_ref[...]
    h = h * jax.nn.sigmoid(h)
    y = jnp.dot(h, w2_ref[...], preferred_element_type=jnp.float32) + b2_ref[...]
    yw = y.reshape(ROWS, 128)
    rows = (
        i * BLK
        + lax.broadcasted_iota(jnp.int32, (ROWS, 128), 0) * 128
        + lax.broadcasted_iota(jnp.int32, (ROWS, 128), 1)
    )
    y_ref[...] = jnp.where(rows < N_ATOMS, yw, 0.0)


def _mlp(x, W1, b1, W2, b2):
    return pl.pallas_call(
        _mlp_body,
        grid=(GRID,),
        in_specs=[
            pl.BlockSpec((BLK, N_IN), lambda i: (i, 0)),
            pl.BlockSpec((N_HIDDEN, N_IN), lambda i: (0, 0)),
            pl.BlockSpec((1, N_HIDDEN), lambda i: (0, 0)),
            pl.BlockSpec((N_HIDDEN, 1), lambda i: (0, 0)),
            pl.BlockSpec((1, 1), lambda i: (0, 0)),
        ],
        out_specs=pl.BlockSpec((ROWS, 128), lambda i: (i, 0)),
        out_shape=jax.ShapeDtypeStruct((N_PAD // 128, 128), jnp.float32),
    )(x, W1.T, b1.reshape(1, N_HIDDEN), W2, b2.reshape(1, 1))


def _sc_segment_sum(y_wide, idx_pad):
    mesh = plsc.VectorSubcoreMesh(core_axis_name="c", subcore_axis_name="s")

    @functools.partial(
        pl.kernel,
        mesh=mesh,
        out_type=jax.ShapeDtypeStruct((NWORKERS, N_MOL), jnp.float32),
        scratch_types=[
            pltpu.VMEM((CROWS + 7, 128), jnp.float32),
            pltpu.VMEM((CHUNK,), jnp.int32),
            pltpu.VMEM((N_MOL,), jnp.float32),
            pltpu.SemaphoreType.DMA,
            pltpu.SemaphoreType.DMA,
        ],
        compiler_params=pltpu.CompilerParams(needs_layout_passes=False),
    )
    def body(y_hbm, idx_hbm, out_hbm, y_v, idx_v, acc_v, sem_y, sem_i):
        wid = lax.axis_index("s") * 2 + lax.axis_index("c")
        row0 = wid * CROWS
        base8 = (row0 // 8) * 8
        delta = row0 - base8
        cp_y = pltpu.async_copy(y_hbm.at[pl.ds(base8, CROWS + 7)], y_v, sem_y)

        zero_i = jnp.zeros((LANES,), jnp.int32)

        @pl.when(wid < NWORKERS - 1)
        def _():
            pltpu.async_copy(
                idx_hbm.at[pl.ds(wid * CHUNK, CHUNK)], idx_v, sem_i
            ).wait()

        @pl.when(wid == NWORKERS - 1)
        def _():
            cp_i = pltpu.async_copy(
                idx_hbm.at[pl.ds((NWORKERS - 1) * CHUNK, TAIL)],
                idx_v.at[pl.ds(0, TAIL)],
                sem_i,
            )

            def zpad_body(k, _):
                idx_v[pl.ds(TAIL + k * LANES, LANES)] = zero_i
                return 0

            lax.fori_loop(0, (CHUNK - TAIL) // LANES, zpad_body, 0)
            cp_i.wait()

        zero = jnp.zeros((LANES,), jnp.float32)

        @plsc.parallel_loop(0, N_MOL // LANES, 1, unroll=8)
        def zero_body(k):
            acc_v[pl.ds(k * LANES, LANES)] = zero

        cp_y.wait()

        @plsc.parallel_loop(0, CROWS, 1, unroll=4)
        def row_body(r):
            for c in range(128 // LANES):
                idx = idx_v[pl.ds(r * 128 + c * LANES, LANES)]
                val = y_v[delta + r, pl.ds(c * LANES, LANES)]
                plsc.addupdate_scatter(acc_v, [idx], val)

        pltpu.sync_copy(acc_v, out_hbm.at[wid])

    return body(y_wide, idx_pad)


def _combine_body(p_ref, o_ref):
    o_ref[...] = jnp.sum(p_ref[...], axis=0, keepdims=True)


def _combine(partials):
    return pl.pallas_call(
        _combine_body,
        out_shape=jax.ShapeDtypeStruct((1, N_MOL), jnp.float32),
    )(partials)


def kernel(scalar_representation, idx_m, W1, b1, W2, b2):
    y_wide = _mlp(scalar_representation, W1, b1, W2, b2)
    partials = _sc_segment_sum(y_wide, idx_m.astype(jnp.int32))
    out = _combine(partials)
    return out.reshape(N_MOL)


# speedup vs baseline: 1.0028x; 1.0028x over previous
import functools

import jax
import jax.numpy as jnp
from jax import lax
from jax.experimental import pallas as pl
from jax.experimental.pallas import tpu as pltpu
from jax.experimental.pallas import tpu_sc as plsc

N_ATOMS = 100000
N_IN = 128
N_HIDDEN = 64
N_MOL = 1024

LANES = 16
NWORKERS = 32
BLK = 20480
GRID = 5
N_PAD = GRID * BLK
CHUNK = N_PAD // NWORKERS
TAIL = N_ATOMS - (NWORKERS - 1) * CHUNK
CROWS = CHUNK // 128
ROWS = BLK // 128


def _mlp_body(x_ref, w1t_ref, b1_ref, w2_ref, b2_ref, y_ref):
    i = pl.program_id(0)
    h = lax.dot_general(
        x_ref[...], w1t_ref[...], (((1,), (1,)), ((), ())),
        preferred_element_type=jnp.float32,
    )
    h = h + b1_ref[...]
    h = h * jax.nn.sigmoid(h)
    y = jnp.dot(h, w2_ref[...], preferred_element_type=jnp.float32) + b2_ref[...]
    yw = y.reshape(ROWS, 128)
    rows = (
        i * BLK
        + lax.broadcasted_iota(jnp.int32, (ROWS, 128), 0) * 128
        + lax.broadcasted_iota(jnp.int32, (ROWS, 128), 1)
    )
    y_ref[...] = jnp.where(rows < N_ATOMS, yw, 0.0)


def _mlp(x, W1, b1, W2, b2):
    return pl.pallas_call(
        _mlp_body,
        grid=(GRID,),
        in_specs=[
            pl.BlockSpec((BLK, N_IN), lambda i: (i, 0)),
            pl.BlockSpec((N_HIDDEN, N_IN), lambda i: (0, 0)),
            pl.BlockSpec((1, N_HIDDEN), lambda i: (0, 0)),
            pl.BlockSpec((N_HIDDEN, 1), lambda i: (0, 0)),
            pl.BlockSpec((1, 1), lambda i: (0, 0)),
        ],
        out_specs=pl.BlockSpec((ROWS, 128), lambda i: (i, 0)),
        out_shape=jax.ShapeDtypeStruct((N_PAD // 128, 128), jnp.float32),
    )(x, W1.T, b1.reshape(1, N_HIDDEN), W2, b2.reshape(1, 1))


def _sc_segment_sum(y_wide, idx_pad):
    mesh = plsc.VectorSubcoreMesh(core_axis_name="c", subcore_axis_name="s")

    @functools.partial(
        pl.kernel,
        mesh=mesh,
        out_type=jax.ShapeDtypeStruct((NWORKERS, N_MOL), jnp.float32),
        scratch_types=[
            pltpu.VMEM((CROWS + 7, 128), jnp.float32),
            pltpu.VMEM((CHUNK,), jnp.int32),
            pltpu.VMEM((N_MOL,), jnp.float32),
            pltpu.SemaphoreType.DMA,
            pltpu.SemaphoreType.DMA,
        ],
        compiler_params=pltpu.CompilerParams(needs_layout_passes=False),
    )
    def body(y_hbm, idx_hbm, out_hbm, y_v, idx_v, acc_v, sem_y, sem_i):
        wid = lax.axis_index("s") * 2 + lax.axis_index("c")
        row0 = wid * CROWS
        base8 = (row0 // 8) * 8
        delta = row0 - base8
        cp_y = pltpu.async_copy(y_hbm.at[pl.ds(base8, CROWS + 7)], y_v, sem_y)

        zero_i = jnp.zeros((LANES,), jnp.int32)

        @pl.when(wid < NWORKERS - 1)
        def _():
            pltpu.async_copy(
                idx_hbm.at[pl.ds(wid * CHUNK, CHUNK)], idx_v, sem_i
            ).wait()

        @pl.when(wid == NWORKERS - 1)
        def _():
            cp_i = pltpu.async_copy(
                idx_hbm.at[pl.ds((NWORKERS - 1) * CHUNK, TAIL)],
                idx_v.at[pl.ds(0, TAIL)],
                sem_i,
            )

            def zpad_body(k, _):
                idx_v[pl.ds(TAIL + k * LANES, LANES)] = zero_i
                return 0

            lax.fori_loop(0, (CHUNK - TAIL) // LANES, zpad_body, 0)
            cp_i.wait()

        zero = jnp.zeros((LANES,), jnp.float32)

        @plsc.parallel_loop(0, N_MOL // LANES, 1, unroll=8)
        def zero_body(k):
            acc_v[pl.ds(k * LANES, LANES)] = zero

        cp_y.wait()

        @plsc.parallel_loop(0, CROWS, 1, unroll=4)
        def row_body(r):
            for c in range(128 // LANES):
                idx = idx_v[pl.ds(r * 128 + c * LANES, LANES)]
                val = y_v[delta + r, pl.ds(c * LANES, LANES)]
                plsc.addupdate_scatter(acc_v, [idx], val)

        pltpu.sync_copy(acc_v, out_hbm.at[wid])

    return body(y_wide, idx_pad)


def _combine_body(p_ref, o_ref):
    o_ref[...] = jnp.sum(p_ref[...], axis=0, keepdims=True)


def _combine(partials):
    return pl.pallas_call(
        _combine_body,
        out_shape=jax.ShapeDtypeStruct((1, N_MOL), jnp.float32),
    )(partials)


def kernel(scalar_representation, idx_m, W1, b1, W2, b2):
    y_wide = _mlp(scalar_representation, W1, b1, W2, b2)
    partials = _sc_segment_sum(y_wide, idx_m.astype(jnp.int32))
    out = _combine(partials)
    return out.reshape(N_MOL)
